# baseline (speedup 1.0000x reference)
import functools

import jax
import jax.numpy as jnp
from jax.experimental import pallas as pl
from jax.experimental.pallas import tpu as pltpu

E = 8
TOPK = 2
D = 2048
FF = 4096
T = 2048

TR = 128
NR = T // TR
TF = 256
NF = FF // TF
TRS = 256
NRS = T // TRS
TFS = 256
NFS = FF // TFS


def _routing_body(x_ref, wg_ref, counts_ref, post_ref, fwt_ref, xb_ref):
    xb_ref[...] = x_ref[...].astype(jnp.bfloat16)
    logits = jax.lax.dot_general(
        wg_ref[...], x_ref[...], (((0,), (1,)), ((), ())),
        preferred_element_type=jnp.float32)
    m = jnp.max(logits, axis=0, keepdims=True)
    p = jnp.exp(logits - m)
    p = p / jnp.sum(p, axis=0, keepdims=True)

    eidx = jax.lax.broadcasted_iota(jnp.int32, (E, T), 0)
    w1 = jnp.max(p, axis=0, keepdims=True)
    i1 = jnp.min(jnp.where(p == w1, eidx, E), axis=0, keepdims=True)
    p2 = jnp.where(eidx == i1, -jnp.inf, p)
    w2 = jnp.max(p2, axis=0, keepdims=True)
    i2 = jnp.min(jnp.where(p2 == w2, eidx, E), axis=0, keepdims=True)
    s = w1 + w2
    w1n = w1 / s
    w2n = w2 / s

    sel1 = (eidx == i1)
    sel2 = (eidx == i2)
    mask = (sel1 | sel2).astype(jnp.float32)
    fwt = jnp.where(sel1, w1n, 0.0) + jnp.where(sel2, w2n, 0.0)

    ti = jax.lax.broadcasted_iota(jnp.int32, (T, T), 0)
    tj = jax.lax.broadcasted_iota(jnp.int32, (T, T), 1)
    upper = (ti < tj).astype(jnp.float32)
    pos = jax.lax.dot_general(
        mask, upper, (((1,), (0,)), ((), ())),
        preferred_element_type=jnp.float32)

    post_ref[...] = jnp.where(mask > 0.0, pos, -1.0)
    fwt_ref[...] = fwt
    counts_ref[...] = jnp.sum(mask, axis=1, keepdims=True).astype(jnp.int32)


def _routing(x, Wg):
    return pl.pallas_call(
        _routing_body,
        out_shape=(
            jax.ShapeDtypeStruct((E, 1), jnp.int32),
            jax.ShapeDtypeStruct((E, T), jnp.float32),
            jax.ShapeDtypeStruct((E, T), jnp.float32),
            jax.ShapeDtypeStruct((T, D), jnp.bfloat16),
        ),
        compiler_params=pltpu.CompilerParams(
            vmem_limit_bytes=100 * 1024 * 1024),
    )(x, Wg)


def _expert_body(counts_s, post_ref, fwt_ref, x_ref, wg_ref, wu_ref, wd_ref,
                 out_ref, acc_ref, xg_ref):
    e = pl.program_id(0)
    f = pl.program_id(1)

    @pl.when((e == 0) & (f == 0))
    def _init():
        out_ref[...] = jnp.zeros_like(out_ref)

    wgb = wg_ref[0].astype(jnp.bfloat16)
    wub = wu_ref[0].astype(jnp.bfloat16)
    wdb = wd_ref[0].astype(jnp.bfloat16)
    pos_row = post_ref[0]
    riota = jax.lax.broadcasted_iota(jnp.int32, (TR, 1), 0)
    nact = (counts_s[e] + TR - 1) // TR

    def tile_body(r, carry):
        rows = pl.ds(r * TR, TR)
        slot = ((r * TR) + riota).astype(jnp.float32)
        gsel = (pos_row == slot)

        @pl.when(f == 0)
        def _gather():
            xg_ref[rows, :] = jax.lax.dot_general(
                gsel.astype(jnp.bfloat16), x_ref[...], (((1,), (0,)), ((), ())),
                preferred_element_type=jnp.float32).astype(jnp.bfloat16)

        xg = xg_ref[rows, :]
        gate = jax.lax.dot_general(
            xg, wgb, (((1,), (0,)), ((), ())),
            preferred_element_type=jnp.float32)
        up = jax.lax.dot_general(
            xg, wub, (((1,), (0,)), ((), ())),
            preferred_element_type=jnp.float32)
        act = (jax.nn.silu(gate) * up).astype(jnp.bfloat16)
        part = jax.lax.dot_general(
            act, wdb, (((1,), (0,)), ((), ())),
            preferred_element_type=jnp.float32)

        @pl.when(f == 0)
        def _():
            acc_ref[rows, :] = part

        @pl.when(f > 0)
        def _():
            acc_ref[rows, :] += part

        @pl.when(f == NF - 1)
        def _scatter():
            g = (jnp.where(gsel, fwt_ref[0], 0.0)).astype(jnp.bfloat16)
            out_ref[...] += jax.lax.dot_general(
                g, acc_ref[rows, :].astype(jnp.bfloat16), (((0,), (0,)), ((), ())),
                preferred_element_type=jnp.float32).astype(jnp.bfloat16)

        return carry

    jax.lax.fori_loop(0, nact, tile_body, 0)


def _expert_block(counts, post, fwt, x, W_gate_up, W_down):
    grid_spec = pltpu.PrefetchScalarGridSpec(
        num_scalar_prefetch=1,
        grid=(E, NF),
        in_specs=[
            pl.BlockSpec((1, 1, T), lambda e, f, c: (e, 0, 0)),
            pl.BlockSpec((1, 1, T), lambda e, f, c: (e, 0, 0)),
            pl.BlockSpec((T, D), lambda e, f, c: (0, 0)),
            pl.BlockSpec((1, D, TF), lambda e, f, c: (e, 0, f)),
            pl.BlockSpec((1, D, TF), lambda e, f, c: (e, 0, f + NF)),
            pl.BlockSpec((1, TF, D), lambda e, f, c: (e, f, 0)),
        ],
        out_specs=pl.BlockSpec((T, D), lambda e, f, c: (0, 0)),
        scratch_shapes=[
            pltpu.VMEM((T, D), jnp.float32),
            pltpu.VMEM((T, D), jnp.bfloat16),
        ],
    )
    return pl.pallas_call(
        _expert_body,
        grid_spec=grid_spec,
        out_shape=jax.ShapeDtypeStruct((T, D), jnp.bfloat16),
        compiler_params=pltpu.CompilerParams(
            dimension_semantics=("arbitrary", "arbitrary"),
            vmem_limit_bytes=63 * 1024 * 1024),
    )(counts, post, fwt, x, W_gate_up, W_gate_up, W_down)


def _shared_body(x_ref, moe_ref, wg_ref, wu_ref, wd_ref, out_ref):
    f = pl.program_id(0)
    wgb = wg_ref[...].astype(jnp.bfloat16)
    wub = wu_ref[...].astype(jnp.bfloat16)
    wdb = wd_ref[...].astype(jnp.bfloat16)
    for r in range(NRS):
        xr = x_ref[r * TRS:(r + 1) * TRS, :]
        gate = jax.lax.dot_general(
            xr, wgb, (((1,), (0,)), ((), ())),
            preferred_element_type=jnp.float32)
        up = jax.lax.dot_general(
            xr, wub, (((1,), (0,)), ((), ())),
            preferred_element_type=jnp.float32)
        act = (jax.nn.silu(gate) * up).astype(jnp.bfloat16)
        part = jax.lax.dot_general(
            act, wdb, (((1,), (0,)), ((), ())),
            preferred_element_type=jnp.float32)

        @pl.when(f == 0)
        def _(part=part, r=r):
            out_ref[r * TRS:(r + 1) * TRS, :] = part + moe_ref[
                r * TRS:(r + 1) * TRS, :].astype(jnp.float32)

        @pl.when(f > 0)
        def _(part=part, r=r):
            out_ref[r * TRS:(r + 1) * TRS, :] += part


def _shared_block(x, moe_out, Ws_gate_up, Ws_down):
    return pl.pallas_call(
        _shared_body,
        grid=(NFS,),
        in_specs=[
            pl.BlockSpec((T, D), lambda f: (0, 0)),
            pl.BlockSpec((T, D), lambda f: (0, 0)),
            pl.BlockSpec((D, TFS), lambda f: (0, f)),
            pl.BlockSpec((D, TFS), lambda f: (0, f + NFS)),
            pl.BlockSpec((TFS, D), lambda f: (f, 0)),
        ],
        out_specs=pl.BlockSpec((T, D), lambda f: (0, 0)),
        out_shape=jax.ShapeDtypeStruct((T, D), jnp.float32),
        compiler_params=pltpu.CompilerParams(
            dimension_semantics=("arbitrary",),
            vmem_limit_bytes=63 * 1024 * 1024),
    )(x, moe_out, Ws_gate_up, Ws_gate_up, Ws_down)


@jax.jit
def _run(hidden_states, Wg, W_gate_up, W_down, Ws_gate_up, Ws_down):
    orig_shape = hidden_states.shape
    x = hidden_states.reshape(-1, D)
    counts, post, fwt, xb = _routing(x, Wg)
    counts = counts.reshape(E)
    post = post.reshape(E, 1, T)
    fwt = fwt.reshape(E, 1, T)
    moe_out = _expert_block(counts, post, fwt, xb, W_gate_up, W_down)
    final = _shared_block(xb, moe_out, Ws_gate_up, Ws_down)
    return final.reshape(orig_shape)


def kernel(hidden_states, Wg, W_gate_up, W_down, Ws_gate_up, Ws_down):
    return _run(hidden_states, Wg, W_gate_up, W_down, Ws_gate_up, Ws_down)


# speedup vs baseline: 1.1768x; 1.1768x over previous
import functools

import jax
import jax.numpy as jnp
from jax.experimental import pallas as pl
from jax.experimental.pallas import tpu as pltpu

E = 8
TOPK = 2
D = 2048
FF = 4096
T = 2048

TR = 256
NR = T // TR
TF = 256
NF = FF // TF
TRS = 256
NRS = T // TRS
TFS = 256
NFS = FF // TFS


def _routing_body(x_ref, wg_ref, counts_ref, post_ref, fwt_ref, xb_ref):
    xb_ref[...] = x_ref[...].astype(jnp.bfloat16)
    logits = jax.lax.dot_general(
        wg_ref[...], x_ref[...], (((0,), (1,)), ((), ())),
        preferred_element_type=jnp.float32)
    m = jnp.max(logits, axis=0, keepdims=True)
    p = jnp.exp(logits - m)
    p = p / jnp.sum(p, axis=0, keepdims=True)

    eidx = jax.lax.broadcasted_iota(jnp.int32, (E, T), 0)
    w1 = jnp.max(p, axis=0, keepdims=True)
    i1 = jnp.min(jnp.where(p == w1, eidx, E), axis=0, keepdims=True)
    p2 = jnp.where(eidx == i1, -jnp.inf, p)
    w2 = jnp.max(p2, axis=0, keepdims=True)
    i2 = jnp.min(jnp.where(p2 == w2, eidx, E), axis=0, keepdims=True)
    s = w1 + w2
    w1n = w1 / s
    w2n = w2 / s

    sel1 = (eidx == i1)
    sel2 = (eidx == i2)
    mask = (sel1 | sel2).astype(jnp.float32)
    fwt = jnp.where(sel1, w1n, 0.0) + jnp.where(sel2, w2n, 0.0)

    ti = jax.lax.broadcasted_iota(jnp.int32, (T, T), 0)
    tj = jax.lax.broadcasted_iota(jnp.int32, (T, T), 1)
    upper = (ti < tj).astype(jnp.float32)
    pos = jax.lax.dot_general(
        mask, upper, (((1,), (0,)), ((), ())),
        preferred_element_type=jnp.float32)

    post_ref[...] = jnp.where(mask > 0.0, pos, -1.0)
    fwt_ref[...] = fwt
    counts_ref[...] = jnp.sum(mask, axis=1, keepdims=True).astype(jnp.int32)


def _routing(x, Wg):
    return pl.pallas_call(
        _routing_body,
        out_shape=(
            jax.ShapeDtypeStruct((E, 1), jnp.int32),
            jax.ShapeDtypeStruct((E, T), jnp.float32),
            jax.ShapeDtypeStruct((E, T), jnp.float32),
            jax.ShapeDtypeStruct((T, D), jnp.bfloat16),
        ),
        compiler_params=pltpu.CompilerParams(
            vmem_limit_bytes=100 * 1024 * 1024),
    )(x, Wg)


def _expert_body(counts_s, post_ref, fwt_ref, x_ref, wg_ref, wu_ref, wd_ref,
                 out_ref, acc_ref, xg_ref):
    e = pl.program_id(0)
    f = pl.program_id(1)

    @pl.when((e == 0) & (f == 0))
    def _init():
        out_ref[...] = jnp.zeros_like(out_ref)

    wgb = wg_ref[0].astype(jnp.bfloat16)
    wub = wu_ref[0].astype(jnp.bfloat16)
    wdb = wd_ref[0].astype(jnp.bfloat16)
    pos_row = post_ref[0]
    riota = jax.lax.broadcasted_iota(jnp.int32, (TR, 1), 0)
    nact = (counts_s[e] + TR - 1) // TR

    def tile_body(r, carry):
        rows = pl.ds(r * TR, TR)
        slot = ((r * TR) + riota).astype(jnp.float32)
        gsel = (pos_row == slot)

        @pl.when(f == 0)
        def _gather():
            xg_ref[rows, :] = jax.lax.dot_general(
                gsel.astype(jnp.bfloat16), x_ref[...], (((1,), (0,)), ((), ())),
                preferred_element_type=jnp.float32).astype(jnp.bfloat16)

        xg = xg_ref[rows, :]
        gate = jax.lax.dot_general(
            xg, wgb, (((1,), (0,)), ((), ())),
            preferred_element_type=jnp.float32)
        up = jax.lax.dot_general(
            xg, wub, (((1,), (0,)), ((), ())),
            preferred_element_type=jnp.float32)
        act = (jax.nn.silu(gate) * up).astype(jnp.bfloat16)
        part = jax.lax.dot_general(
            act, wdb, (((1,), (0,)), ((), ())),
            preferred_element_type=jnp.float32)

        @pl.when(f == 0)
        def _():
            acc_ref[rows, :] = part

        @pl.when(f > 0)
        def _():
            acc_ref[rows, :] += part

        @pl.when(f == NF - 1)
        def _scatter():
            g = (jnp.where(gsel, fwt_ref[0], 0.0)).astype(jnp.bfloat16)
            out_ref[...] += jax.lax.dot_general(
                g, acc_ref[rows, :].astype(jnp.bfloat16), (((0,), (0,)), ((), ())),
                preferred_element_type=jnp.float32).astype(jnp.bfloat16)

        return carry

    jax.lax.fori_loop(0, nact, tile_body, 0)


def _expert_block(counts, post, fwt, x, W_gate_up, W_down):
    grid_spec = pltpu.PrefetchScalarGridSpec(
        num_scalar_prefetch=1,
        grid=(E, NF),
        in_specs=[
            pl.BlockSpec((1, 1, T), lambda e, f, c: (e, 0, 0)),
            pl.BlockSpec((1, 1, T), lambda e, f, c: (e, 0, 0)),
            pl.BlockSpec((T, D), lambda e, f, c: (0, 0)),
            pl.BlockSpec((1, D, TF), lambda e, f, c: (e, 0, f)),
            pl.BlockSpec((1, D, TF), lambda e, f, c: (e, 0, f + NF)),
            pl.BlockSpec((1, TF, D), lambda e, f, c: (e, f, 0)),
        ],
        out_specs=pl.BlockSpec((T, D), lambda e, f, c: (0, 0)),
        scratch_shapes=[
            pltpu.VMEM((T, D), jnp.float32),
            pltpu.VMEM((T, D), jnp.bfloat16),
        ],
    )
    return pl.pallas_call(
        _expert_body,
        grid_spec=grid_spec,
        out_shape=jax.ShapeDtypeStruct((T, D), jnp.bfloat16),
        compiler_params=pltpu.CompilerParams(
            dimension_semantics=("arbitrary", "arbitrary"),
            vmem_limit_bytes=63 * 1024 * 1024),
    )(counts, post, fwt, x, W_gate_up, W_gate_up, W_down)


def _shared_body(x_ref, moe_ref, wg_ref, wu_ref, wd_ref, out_ref):
    f = pl.program_id(0)
    wgb = wg_ref[...].astype(jnp.bfloat16)
    wub = wu_ref[...].astype(jnp.bfloat16)
    wdb = wd_ref[...].astype(jnp.bfloat16)
    for r in range(NRS):
        xr = x_ref[r * TRS:(r + 1) * TRS, :]
        gate = jax.lax.dot_general(
            xr, wgb, (((1,), (0,)), ((), ())),
            preferred_element_type=jnp.float32)
        up = jax.lax.dot_general(
            xr, wub, (((1,), (0,)), ((), ())),
            preferred_element_type=jnp.float32)
        act = (jax.nn.silu(gate) * up).astype(jnp.bfloat16)
        part = jax.lax.dot_general(
            act, wdb, (((1,), (0,)), ((), ())),
            preferred_element_type=jnp.float32)

        @pl.when(f == 0)
        def _(part=part, r=r):
            out_ref[r * TRS:(r + 1) * TRS, :] = part + moe_ref[
                r * TRS:(r + 1) * TRS, :].astype(jnp.float32)

        @pl.when(f > 0)
        def _(part=part, r=r):
            out_ref[r * TRS:(r + 1) * TRS, :] += part


def _shared_block(x, moe_out, Ws_gate_up, Ws_down):
    return pl.pallas_call(
        _shared_body,
        grid=(NFS,),
        in_specs=[
            pl.BlockSpec((T, D), lambda f: (0, 0)),
            pl.BlockSpec((T, D), lambda f: (0, 0)),
            pl.BlockSpec((D, TFS), lambda f: (0, f)),
            pl.BlockSpec((D, TFS), lambda f: (0, f + NFS)),
            pl.BlockSpec((TFS, D), lambda f: (f, 0)),
        ],
        out_specs=pl.BlockSpec((T, D), lambda f: (0, 0)),
        out_shape=jax.ShapeDtypeStruct((T, D), jnp.float32),
        compiler_params=pltpu.CompilerParams(
            dimension_semantics=("arbitrary",),
            vmem_limit_bytes=63 * 1024 * 1024),
    )(x, moe_out, Ws_gate_up, Ws_gate_up, Ws_down)


@jax.jit
def _run(hidden_states, Wg, W_gate_up, W_down, Ws_gate_up, Ws_down):
    orig_shape = hidden_states.shape
    x = hidden_states.reshape(-1, D)
    counts, post, fwt, xb = _routing(x, Wg)
    counts = counts.reshape(E)
    post = post.reshape(E, 1, T)
    fwt = fwt.reshape(E, 1, T)
    moe_out = _expert_block(counts, post, fwt, xb, W_gate_up, W_down)
    final = _shared_block(xb, moe_out, Ws_gate_up, Ws_down)
    return final.reshape(orig_shape)


def kernel(hidden_states, Wg, W_gate_up, W_down, Ws_gate_up, Ws_down):
    return _run(hidden_states, Wg, W_gate_up, W_down, Ws_gate_up, Ws_down)


# speedup vs baseline: 1.2451x; 1.0581x over previous
import functools

import jax
import jax.numpy as jnp
from jax.experimental import pallas as pl
from jax.experimental.pallas import tpu as pltpu

E = 8
TOPK = 2
D = 2048
FF = 4096
T = 2048

TR = 256
NR = T // TR
TF = 256
NF = FF // TF
TRS = 256
NRS = T // TRS
TFS = 512
NFS = FF // TFS


def _routing_body(x_ref, wg_ref, counts_ref, post_ref, fwt_ref, xb_ref):
    xb_ref[...] = x_ref[...].astype(jnp.bfloat16)
    logits = jax.lax.dot_general(
        wg_ref[...], x_ref[...], (((0,), (1,)), ((), ())),
        preferred_element_type=jnp.float32)
    m = jnp.max(logits, axis=0, keepdims=True)
    p = jnp.exp(logits - m)
    p = p / jnp.sum(p, axis=0, keepdims=True)

    eidx = jax.lax.broadcasted_iota(jnp.int32, (E, T), 0)
    w1 = jnp.max(p, axis=0, keepdims=True)
    i1 = jnp.min(jnp.where(p == w1, eidx, E), axis=0, keepdims=True)
    p2 = jnp.where(eidx == i1, -jnp.inf, p)
    w2 = jnp.max(p2, axis=0, keepdims=True)
    i2 = jnp.min(jnp.where(p2 == w2, eidx, E), axis=0, keepdims=True)
    s = w1 + w2
    w1n = w1 / s
    w2n = w2 / s

    sel1 = (eidx == i1)
    sel2 = (eidx == i2)
    mask = (sel1 | sel2).astype(jnp.float32)
    fwt = jnp.where(sel1, w1n, 0.0) + jnp.where(sel2, w2n, 0.0)

    ti = jax.lax.broadcasted_iota(jnp.int32, (T, T), 0)
    tj = jax.lax.broadcasted_iota(jnp.int32, (T, T), 1)
    upper = (ti < tj).astype(jnp.float32)
    pos = jax.lax.dot_general(
        mask, upper, (((1,), (0,)), ((), ())),
        preferred_element_type=jnp.float32)

    post_ref[...] = jnp.where(mask > 0.0, pos, -1.0)
    fwt_ref[...] = fwt
    counts_ref[...] = jnp.sum(mask, axis=1, keepdims=True).astype(jnp.int32)


def _routing(x, Wg):
    return pl.pallas_call(
        _routing_body,
        out_shape=(
            jax.ShapeDtypeStruct((E, 1), jnp.int32),
            jax.ShapeDtypeStruct((E, T), jnp.float32),
            jax.ShapeDtypeStruct((E, T), jnp.float32),
            jax.ShapeDtypeStruct((T, D), jnp.bfloat16),
        ),
        compiler_params=pltpu.CompilerParams(
            vmem_limit_bytes=100 * 1024 * 1024),
    )(x, Wg)


def _expert_body(counts_s, post_ref, fwt_ref, x_ref, wg_ref, wu_ref, wd_ref,
                 out_ref, acc_ref, xg_ref):
    e = pl.program_id(0)
    f = pl.program_id(1)

    @pl.when((e == 0) & (f == 0))
    def _init():
        out_ref[...] = jnp.zeros_like(out_ref)

    wgb = wg_ref[0].astype(jnp.bfloat16)
    wub = wu_ref[0].astype(jnp.bfloat16)
    wdb = wd_ref[0].astype(jnp.bfloat16)
    pos_row = post_ref[0]
    riota = jax.lax.broadcasted_iota(jnp.int32, (TR, 1), 0)
    nact = (counts_s[e] + TR - 1) // TR

    def tile_body(r, carry):
        rows = pl.ds(r * TR, TR)

        def _gsel():
            slot = ((r * TR) + riota).astype(jnp.float32)
            return pos_row == slot

        @pl.when(f == 0)
        def _gather():
            xg_ref[rows, :] = jax.lax.dot_general(
                _gsel().astype(jnp.bfloat16), x_ref[...],
                (((1,), (0,)), ((), ())),
                preferred_element_type=jnp.float32).astype(jnp.bfloat16)

        xg = xg_ref[rows, :]
        gate = jax.lax.dot_general(
            xg, wgb, (((1,), (0,)), ((), ())),
            preferred_element_type=jnp.float32)
        up = jax.lax.dot_general(
            xg, wub, (((1,), (0,)), ((), ())),
            preferred_element_type=jnp.float32)
        act = (jax.nn.silu(gate) * up).astype(jnp.bfloat16)
        part = jax.lax.dot_general(
            act, wdb, (((1,), (0,)), ((), ())),
            preferred_element_type=jnp.float32)

        @pl.when(f == 0)
        def _():
            acc_ref[rows, :] = part.astype(jnp.bfloat16)

        @pl.when(f > 0)
        def _():
            acc_ref[rows, :] += part.astype(jnp.bfloat16)

        @pl.when(f == NF - 1)
        def _scatter():
            g = (jnp.where(_gsel(), fwt_ref[0], 0.0)).astype(jnp.bfloat16)
            out_ref[...] += jax.lax.dot_general(
                g, acc_ref[rows, :], (((0,), (0,)), ((), ())),
                preferred_element_type=jnp.float32).astype(jnp.bfloat16)

        return carry

    jax.lax.fori_loop(0, nact, tile_body, 0)


def _expert_block(counts, post, fwt, x, W_gate_up, W_down):
    grid_spec = pltpu.PrefetchScalarGridSpec(
        num_scalar_prefetch=1,
        grid=(E, NF),
        in_specs=[
            pl.BlockSpec((1, 1, T), lambda e, f, c: (e, 0, 0)),
            pl.BlockSpec((1, 1, T), lambda e, f, c: (e, 0, 0)),
            pl.BlockSpec((T, D), lambda e, f, c: (0, 0)),
            pl.BlockSpec((1, D, TF), lambda e, f, c: (e, 0, f)),
            pl.BlockSpec((1, D, TF), lambda e, f, c: (e, 0, f + NF)),
            pl.BlockSpec((1, TF, D), lambda e, f, c: (e, f, 0)),
        ],
        out_specs=pl.BlockSpec((T, D), lambda e, f, c: (0, 0)),
        scratch_shapes=[
            pltpu.VMEM((T, D), jnp.bfloat16),
            pltpu.VMEM((T, D), jnp.bfloat16),
        ],
    )
    return pl.pallas_call(
        _expert_body,
        grid_spec=grid_spec,
        out_shape=jax.ShapeDtypeStruct((T, D), jnp.bfloat16),
        compiler_params=pltpu.CompilerParams(
            dimension_semantics=("arbitrary", "arbitrary"),
            vmem_limit_bytes=63 * 1024 * 1024),
    )(counts, post, fwt, x, W_gate_up, W_gate_up, W_down)


def _shared_body(x_ref, wg_ref, wu_ref, wd_ref, out_ref):
    f = pl.program_id(0)
    wgb = wg_ref[...].astype(jnp.bfloat16)
    wub = wu_ref[...].astype(jnp.bfloat16)
    wdb = wd_ref[...].astype(jnp.bfloat16)
    for r in range(NRS):
        xr = x_ref[r * TRS:(r + 1) * TRS, :]
        gate = jax.lax.dot_general(
            xr, wgb, (((1,), (0,)), ((), ())),
            preferred_element_type=jnp.float32)
        up = jax.lax.dot_general(
            xr, wub, (((1,), (0,)), ((), ())),
            preferred_element_type=jnp.float32)
        act = (jax.nn.silu(gate) * up).astype(jnp.bfloat16)
        part = jax.lax.dot_general(
            act, wdb, (((1,), (0,)), ((), ())),
            preferred_element_type=jnp.float32)

        @pl.when(f == 0)
        def _(part=part, r=r):
            out_ref[r * TRS:(r + 1) * TRS, :] = part

        @pl.when(f > 0)
        def _(part=part, r=r):
            out_ref[r * TRS:(r + 1) * TRS, :] += part


def _shared_block(x, Ws_gate_up, Ws_down):
    return pl.pallas_call(
        _shared_body,
        grid=(NFS,),
        in_specs=[
            pl.BlockSpec((T, D), lambda f: (0, 0)),
            pl.BlockSpec((D, TFS), lambda f: (0, f)),
            pl.BlockSpec((D, TFS), lambda f: (0, f + NFS)),
            pl.BlockSpec((TFS, D), lambda f: (f, 0)),
        ],
        out_specs=pl.BlockSpec((T, D), lambda f: (0, 0)),
        out_shape=jax.ShapeDtypeStruct((T, D), jnp.float32),
        compiler_params=pltpu.CompilerParams(
            dimension_semantics=("arbitrary",),
            vmem_limit_bytes=63 * 1024 * 1024),
    )(x, Ws_gate_up, Ws_gate_up, Ws_down)


@jax.jit
def _run(hidden_states, Wg, W_gate_up, W_down, Ws_gate_up, Ws_down):
    orig_shape = hidden_states.shape
    x = hidden_states.reshape(-1, D)
    counts, post, fwt, xb = _routing(x, Wg)
    counts = counts.reshape(E)
    post = post.reshape(E, 1, T)
    fwt = fwt.reshape(E, 1, T)
    moe_out = _expert_block(counts, post, fwt, xb, W_gate_up, W_down)
    shared = _shared_block(xb, Ws_gate_up, Ws_down)
    return (moe_out.astype(jnp.float32) + shared).reshape(orig_shape)


def kernel(hidden_states, Wg, W_gate_up, W_down, Ws_gate_up, Ws_down):
    return _run(hidden_states, Wg, W_gate_up, W_down, Ws_gate_up, Ws_down)
